# baseline (speedup 1.0000x reference)
import functools

import jax
import jax.numpy as jnp
from jax import lax
from jax.experimental import pallas as pl
from jax.experimental.pallas import tpu as pltpu
from jax.experimental.pallas import tpu_sc as plsc

D_MODEL = 768
LANES = 16
NUM_CORES = 2
NUM_SUBCORES = 16
NW = NUM_CORES * NUM_SUBCORES
CH = 32
NBUF = 2


def _make_emb_kernel(n_tot: int):
    per_w = n_tot // NW
    steps = per_w // CH
    mesh = plsc.VectorSubcoreMesh(core_axis_name="c", subcore_axis_name="s")

    @functools.partial(
        pl.kernel,
        mesh=mesh,
        out_type=jax.ShapeDtypeStruct((n_tot, D_MODEL), jnp.float32),
        scratch_types=[
            pltpu.VMEM((NBUF, CH), jnp.int32),
            pltpu.VMEM((NBUF, CH), jnp.int32),
            pltpu.VMEM((NBUF, CH, D_MODEL), jnp.float32),
            pltpu.VMEM((NBUF, CH, D_MODEL), jnp.float32),
            pltpu.SemaphoreType.DMA((NBUF,)),
            pltpu.SemaphoreType.DMA((NBUF,)),
            pltpu.SemaphoreType.DMA((NBUF,)),
        ],
    )
    def emb(x_hbm, p_hbm, tok_hbm, posw_hbm, out_hbm,
            xidx, pidx, arows, brows, sem_a, sem_b, sem_o):
        wid = lax.axis_index("s") * NUM_CORES + lax.axis_index("c")
        base = wid * per_w

        def fetch(g, p):
            off = base + g * CH
            pltpu.sync_copy(x_hbm.at[pl.ds(off, CH)], xidx.at[p])
            pltpu.sync_copy(p_hbm.at[pl.ds(off, CH)], pidx.at[p])
            pltpu.async_copy(tok_hbm.at[xidx.at[p]], arows.at[p], sem_a.at[p])
            pltpu.async_copy(posw_hbm.at[pidx.at[p]], brows.at[p], sem_b.at[p])

        def wait_gathers(p):
            pltpu.make_async_copy(tok_hbm.at[xidx.at[p]], arows.at[p],
                                  sem_a.at[p]).wait()
            pltpu.make_async_copy(posw_hbm.at[pidx.at[p]], brows.at[p],
                                  sem_b.at[p]).wait()

        def wait_out(p, g):
            off = base + g * CH
            pltpu.make_async_copy(arows.at[p], out_hbm.at[pl.ds(off, CH)],
                                  sem_o.at[p]).wait()

        fetch(0, 0)

        def pair(go, carry):
            for p in range(NBUF):
                g = go * NBUF + p
                q = (p + 1) % NBUF

                @pl.when(g + 1 < steps)
                def _():
                    @pl.when(g >= 1)
                    def _():
                        wait_out(q, g - 1)
                    fetch(g + 1, q)

                wait_gathers(p)

                def row(r, rcarry):
                    for cc in range(D_MODEL // LANES):
                        sl = pl.ds(cc * LANES, LANES)
                        plsc.addupdate(arows.at[p, r, sl], brows[p, r, sl])
                    return rcarry

                lax.fori_loop(0, CH, row, 0, unroll=False)

                off = base + g * CH
                pltpu.async_copy(arows.at[p], out_hbm.at[pl.ds(off, CH)],
                                 sem_o.at[p])
            return carry

        lax.fori_loop(0, steps // NBUF, pair, 0, unroll=False)
        for p in range(NBUF):
            wait_out(p, steps - NBUF + p)

    return emb


def kernel(x, pos, token_weight, pos_weight):
    orig_shape = x.shape
    xf = x.reshape(-1).astype(jnp.int32)
    pf = pos.reshape(-1).astype(jnp.int32)
    out = _make_emb_kernel(xf.shape[0])(xf, pf, token_weight, pos_weight)
    return out.reshape(orig_shape + (D_MODEL,))


# speedup vs baseline: 1.0897x; 1.0897x over previous
import functools

import jax
import jax.numpy as jnp
from jax import lax
from jax.experimental import pallas as pl
from jax.experimental.pallas import tpu as pltpu
from jax.experimental.pallas import tpu_sc as plsc

D_MODEL = 768
LANES = 16
NUM_CORES = 2
NUM_SUBCORES = 16
NW = NUM_CORES * NUM_SUBCORES
CH = 32
NBUF = 2


def _make_emb_kernel(n_tot: int):
    per_w = n_tot // NW
    steps = per_w // CH
    mesh = plsc.VectorSubcoreMesh(core_axis_name="c", subcore_axis_name="s")

    @functools.partial(
        pl.kernel,
        mesh=mesh,
        out_type=jax.ShapeDtypeStruct((n_tot, D_MODEL), jnp.float32),
        scratch_types=[
            pltpu.VMEM((per_w,), jnp.int32),
            pltpu.VMEM((per_w,), jnp.int32),
            pltpu.VMEM((NBUF, CH, D_MODEL), jnp.float32),
            pltpu.VMEM((NBUF, CH, D_MODEL), jnp.float32),
            pltpu.SemaphoreType.DMA((NBUF,)),
            pltpu.SemaphoreType.DMA((NBUF,)),
            pltpu.SemaphoreType.DMA((NBUF,)),
        ],
    )
    def emb(x_hbm, p_hbm, tok_hbm, posw_hbm, out_hbm,
            xidx, pidx, arows, brows, sem_a, sem_b, sem_o):
        wid = lax.axis_index("s") * NUM_CORES + lax.axis_index("c")
        base = wid * per_w

        pltpu.sync_copy(x_hbm.at[pl.ds(base, per_w)], xidx)
        pltpu.sync_copy(p_hbm.at[pl.ds(base, per_w)], pidx)

        def fetch(g, p):
            sl = pl.ds(g * CH, CH)
            pltpu.async_copy(tok_hbm.at[xidx.at[sl]], arows.at[p], sem_a.at[p])
            pltpu.async_copy(posw_hbm.at[pidx.at[sl]], brows.at[p], sem_b.at[p])

        def wait_gathers(g, p):
            sl = pl.ds(g * CH, CH)
            pltpu.make_async_copy(tok_hbm.at[xidx.at[sl]], arows.at[p],
                                  sem_a.at[p]).wait()
            pltpu.make_async_copy(posw_hbm.at[pidx.at[sl]], brows.at[p],
                                  sem_b.at[p]).wait()

        def wait_out(p, g):
            off = base + g * CH
            pltpu.make_async_copy(arows.at[p], out_hbm.at[pl.ds(off, CH)],
                                  sem_o.at[p]).wait()

        fetch(0, 0)

        def pair(go, carry):
            for p in range(NBUF):
                g = go * NBUF + p
                q = (p + 1) % NBUF

                @pl.when(g + 1 < steps)
                def _():
                    @pl.when(g >= 1)
                    def _():
                        wait_out(q, g - 1)
                    fetch(g + 1, q)

                wait_gathers(g, p)

                def row(r, rcarry):
                    for cc in range(D_MODEL // LANES):
                        sl = pl.ds(cc * LANES, LANES)
                        plsc.addupdate(arows.at[p, r, sl], brows[p, r, sl])
                    return rcarry

                lax.fori_loop(0, CH, row, 0, unroll=False)

                off = base + g * CH
                pltpu.async_copy(arows.at[p], out_hbm.at[pl.ds(off, CH)],
                                 sem_o.at[p])
            return carry

        lax.fori_loop(0, steps // NBUF, pair, 0, unroll=False)
        for p in range(NBUF):
            wait_out(p, steps - NBUF + p)

    return emb


def kernel(x, pos, token_weight, pos_weight):
    orig_shape = x.shape
    xf = x.reshape(-1).astype(jnp.int32)
    pf = pos.reshape(-1).astype(jnp.int32)
    out = _make_emb_kernel(xf.shape[0])(xf, pf, token_weight, pos_weight)
    return out.reshape(orig_shape + (D_MODEL,))


# speedup vs baseline: 1.1522x; 1.0574x over previous
import functools

import jax
import jax.numpy as jnp
from jax import lax
from jax.experimental import pallas as pl
from jax.experimental.pallas import tpu as pltpu
from jax.experimental.pallas import tpu_sc as plsc

D_MODEL = 768
LANES = 16
NUM_CORES = 2
NUM_SUBCORES = 16
NW = NUM_CORES * NUM_SUBCORES
CH = 16
NBUF = 4


def _make_emb_kernel(n_tot: int):
    per_w = n_tot // NW
    steps = per_w // CH
    mesh = plsc.VectorSubcoreMesh(core_axis_name="c", subcore_axis_name="s")

    @functools.partial(
        pl.kernel,
        mesh=mesh,
        out_type=jax.ShapeDtypeStruct((n_tot, D_MODEL), jnp.float32),
        scratch_types=[
            pltpu.VMEM((per_w,), jnp.int32),
            pltpu.VMEM((per_w,), jnp.int32),
            pltpu.VMEM((NBUF, CH, D_MODEL), jnp.float32),
            pltpu.VMEM((NBUF, CH, D_MODEL), jnp.float32),
            pltpu.SemaphoreType.DMA((NBUF,)),
            pltpu.SemaphoreType.DMA((NBUF,)),
            pltpu.SemaphoreType.DMA((NBUF,)),
        ],
    )
    def emb(x_hbm, p_hbm, tok_hbm, posw_hbm, out_hbm,
            xidx, pidx, arows, brows, sem_a, sem_b, sem_o):
        wid = lax.axis_index("s") * NUM_CORES + lax.axis_index("c")
        base = wid * per_w

        pltpu.sync_copy(x_hbm.at[pl.ds(base, per_w)], xidx)
        pltpu.sync_copy(p_hbm.at[pl.ds(base, per_w)], pidx)

        def fetch(g, p):
            sl = pl.ds(g * CH, CH)
            pltpu.async_copy(tok_hbm.at[xidx.at[sl]], arows.at[p], sem_a.at[p])
            pltpu.async_copy(posw_hbm.at[pidx.at[sl]], brows.at[p], sem_b.at[p])

        def wait_gathers(g, p):
            sl = pl.ds(g * CH, CH)
            pltpu.make_async_copy(tok_hbm.at[xidx.at[sl]], arows.at[p],
                                  sem_a.at[p]).wait()
            pltpu.make_async_copy(posw_hbm.at[pidx.at[sl]], brows.at[p],
                                  sem_b.at[p]).wait()

        def wait_out(p, g):
            off = base + g * CH
            pltpu.make_async_copy(arows.at[p], out_hbm.at[pl.ds(off, CH)],
                                  sem_o.at[p]).wait()

        for k in range(NBUF - 1):
            fetch(k, k)

        def phase_group(go, carry):
            for p in range(NBUF):
                g = go * NBUF + p
                f = g + NBUF - 1
                fb = (p + NBUF - 1) % NBUF

                @pl.when(f < steps)
                def _():
                    @pl.when(g >= 1)
                    def _():
                        wait_out(fb, f - NBUF)
                    fetch(f, fb)

                wait_gathers(g, p)

                def row(r, rcarry):
                    for cc in range(D_MODEL // LANES):
                        sl = pl.ds(cc * LANES, LANES)
                        plsc.addupdate(arows.at[p, r, sl], brows[p, r, sl])
                    return rcarry

                lax.fori_loop(0, CH, row, 0, unroll=False)

                off = base + g * CH
                pltpu.async_copy(arows.at[p], out_hbm.at[pl.ds(off, CH)],
                                 sem_o.at[p])
            return carry

        lax.fori_loop(0, steps // NBUF, phase_group, 0, unroll=False)
        for p in range(NBUF):
            wait_out(p, steps - NBUF + p)

    return emb


def kernel(x, pos, token_weight, pos_weight):
    orig_shape = x.shape
    xf = x.reshape(-1).astype(jnp.int32)
    pf = pos.reshape(-1).astype(jnp.int32)
    out = _make_emb_kernel(xf.shape[0])(xf, pf, token_weight, pos_weight)
    return out.reshape(orig_shape + (D_MODEL,))


# speedup vs baseline: 1.2362x; 1.0729x over previous
import functools

import jax
import jax.numpy as jnp
from jax import lax
from jax.experimental import pallas as pl
from jax.experimental.pallas import tpu as pltpu
from jax.experimental.pallas import tpu_sc as plsc

D_MODEL = 768
LANES = 16
NUM_CORES = 2
NUM_SUBCORES = 16
NW = NUM_CORES * NUM_SUBCORES
CH = 16
NBUF = 4


def _make_emb_kernel(n_tot: int):
    per_w = n_tot // NW
    steps = per_w // CH
    mesh = plsc.VectorSubcoreMesh(core_axis_name="c", subcore_axis_name="s")

    @functools.partial(
        pl.kernel,
        mesh=mesh,
        out_type=jax.ShapeDtypeStruct((n_tot, D_MODEL), jnp.float32),
        scratch_types=[
            pltpu.VMEM((per_w,), jnp.int32),
            pltpu.VMEM((per_w,), jnp.int32),
            pltpu.VMEM((NBUF, CH, D_MODEL), jnp.float32),
            pltpu.VMEM((NBUF, CH, D_MODEL), jnp.float32),
            pltpu.SemaphoreType.DMA((NBUF,)),
            pltpu.SemaphoreType.DMA((NBUF,)),
            pltpu.SemaphoreType.DMA((NBUF,)),
        ],
    )
    def emb(x_hbm, p_hbm, tok_hbm, posw_hbm, out_hbm,
            xidx, pidx, arows, brows, sem_a, sem_b, sem_o):
        wid = lax.axis_index("s") * NUM_CORES + lax.axis_index("c")
        base = wid * per_w

        pltpu.sync_copy(x_hbm.at[pl.ds(base, per_w)], xidx)
        pltpu.sync_copy(p_hbm.at[pl.ds(base, per_w)], pidx)

        def fetch_tok(g, p):
            sl = pl.ds(g * CH, CH)
            pltpu.async_copy(tok_hbm.at[xidx.at[sl]], arows.at[p], sem_a.at[p])

        def fetch_pos(g, p):
            sl = pl.ds(g * CH, CH)
            pltpu.async_copy(posw_hbm.at[pidx.at[sl]], brows.at[p], sem_b.at[p])

        def wait_gathers(g, p):
            sl = pl.ds(g * CH, CH)
            pltpu.make_async_copy(tok_hbm.at[xidx.at[sl]], arows.at[p],
                                  sem_a.at[p]).wait()
            pltpu.make_async_copy(posw_hbm.at[pidx.at[sl]], brows.at[p],
                                  sem_b.at[p]).wait()

        def wait_out(p, g):
            off = base + g * CH
            pltpu.make_async_copy(arows.at[p], out_hbm.at[pl.ds(off, CH)],
                                  sem_o.at[p]).wait()

        for k in range(NBUF - 1):
            fetch_tok(k, k)
            fetch_pos(k, k)

        def add_rows(p, lo, hi):
            def row(r, rcarry):
                for cc in range(D_MODEL // LANES):
                    sl = pl.ds(cc * LANES, LANES)
                    plsc.addupdate(arows.at[p, r, sl], brows[p, r, sl])
                return rcarry

            lax.fori_loop(lo, hi, row, 0, unroll=False)

        def phase_group(go, carry):
            for p in range(NBUF):
                g = go * NBUF + p
                f = g + NBUF - 1
                fb = (p + NBUF - 1) % NBUF

                @pl.when(f < steps)
                def _():
                    @pl.when(g >= 1)
                    def _():
                        wait_out(fb, f - NBUF)
                    fetch_tok(f, fb)

                wait_gathers(g, p)
                add_rows(p, 0, CH // 2)

                @pl.when(f < steps)
                def _():
                    fetch_pos(f, fb)

                add_rows(p, CH // 2, CH)

                off = base + g * CH
                pltpu.async_copy(arows.at[p], out_hbm.at[pl.ds(off, CH)],
                                 sem_o.at[p])
            return carry

        lax.fori_loop(0, steps // NBUF, phase_group, 0, unroll=False)
        for p in range(NBUF):
            wait_out(p, steps - NBUF + p)

    return emb


def kernel(x, pos, token_weight, pos_weight):
    orig_shape = x.shape
    xf = x.reshape(-1).astype(jnp.int32)
    pf = pos.reshape(-1).astype(jnp.int32)
    out = _make_emb_kernel(xf.shape[0])(xf, pf, token_weight, pos_weight)
    return out.reshape(orig_shape + (D_MODEL,))


# speedup vs baseline: 1.3774x; 1.1141x over previous
import functools

import jax
import jax.numpy as jnp
from jax import lax
from jax.experimental import pallas as pl
from jax.experimental.pallas import tpu as pltpu
from jax.experimental.pallas import tpu_sc as plsc

D_MODEL = 768
LANES = 16
NUM_CORES = 2
NUM_SUBCORES = 16
NW = NUM_CORES * NUM_SUBCORES
CH = 16
NBUF = 4
PF = 2


def _make_emb_kernel(n_tot: int):
    per_w = n_tot // NW
    steps = per_w // CH
    mesh = plsc.VectorSubcoreMesh(core_axis_name="c", subcore_axis_name="s")

    @functools.partial(
        pl.kernel,
        mesh=mesh,
        out_type=jax.ShapeDtypeStruct((n_tot, D_MODEL), jnp.float32),
        scratch_types=[
            pltpu.VMEM((per_w,), jnp.int32),
            pltpu.VMEM((per_w,), jnp.int32),
            pltpu.VMEM((NBUF, CH, D_MODEL), jnp.float32),
            pltpu.VMEM((NBUF, CH, D_MODEL), jnp.float32),
            pltpu.SemaphoreType.DMA((NBUF,)),
            pltpu.SemaphoreType.DMA((NBUF,)),
            pltpu.SemaphoreType.DMA((NBUF,)),
        ],
    )
    def emb(x_hbm, p_hbm, tok_hbm, posw_hbm, out_hbm,
            xidx, pidx, arows, brows, sem_a, sem_b, sem_o):
        wid = lax.axis_index("s") * NUM_CORES + lax.axis_index("c")
        base = wid * per_w

        pltpu.sync_copy(x_hbm.at[pl.ds(base, per_w)], xidx)
        pltpu.sync_copy(p_hbm.at[pl.ds(base, per_w)], pidx)

        def fetch_tok(g, p):
            sl = pl.ds(g * CH, CH)
            pltpu.async_copy(tok_hbm.at[xidx.at[sl]], arows.at[p], sem_a.at[p])

        def fetch_pos(g, p):
            sl = pl.ds(g * CH, CH)
            pltpu.async_copy(posw_hbm.at[pidx.at[sl]], brows.at[p], sem_b.at[p])

        def wait_gathers(g, p):
            sl = pl.ds(g * CH, CH)
            pltpu.make_async_copy(tok_hbm.at[xidx.at[sl]], arows.at[p],
                                  sem_a.at[p]).wait()
            pltpu.make_async_copy(posw_hbm.at[pidx.at[sl]], brows.at[p],
                                  sem_b.at[p]).wait()

        def wait_out(p, g):
            off = base + g * CH
            pltpu.make_async_copy(arows.at[p], out_hbm.at[pl.ds(off, CH)],
                                  sem_o.at[p]).wait()

        for k in range(PF):
            fetch_tok(k, k)
            fetch_pos(k, k)

        def add_rows(p, lo, hi):
            def row(r, rcarry):
                for cc in range(D_MODEL // LANES):
                    sl = pl.ds(cc * LANES, LANES)
                    plsc.addupdate(arows.at[p, r, sl], brows[p, r, sl])
                return rcarry

            lax.fori_loop(lo, hi, row, 0, unroll=False)

        def phase_group(go, carry):
            for p in range(NBUF):
                g = go * NBUF + p
                f = g + PF
                fb = (p + PF) % NBUF

                @pl.when(f < steps)
                def _():
                    @pl.when(f >= NBUF)
                    def _():
                        wait_out(fb, f - NBUF)
                    fetch_tok(f, fb)

                wait_gathers(g, p)
                add_rows(p, 0, CH // 2)

                @pl.when(f < steps)
                def _():
                    fetch_pos(f, fb)

                add_rows(p, CH // 2, CH)

                off = base + g * CH
                pltpu.async_copy(arows.at[p], out_hbm.at[pl.ds(off, CH)],
                                 sem_o.at[p])
            return carry

        lax.fori_loop(0, steps // NBUF, phase_group, 0, unroll=False)
        for p in range(NBUF):
            wait_out(p, steps - NBUF + p)

    return emb


def kernel(x, pos, token_weight, pos_weight):
    orig_shape = x.shape
    xf = x.reshape(-1).astype(jnp.int32)
    pf = pos.reshape(-1).astype(jnp.int32)
    out = _make_emb_kernel(xf.shape[0])(xf, pf, token_weight, pos_weight)
    return out.reshape(orig_shape + (D_MODEL,))
